# Initial kernel scaffold; baseline (speedup 1.0000x reference)
import jax
import jax.numpy as jnp
from jax.experimental import pallas as pl


def kernel(z):
    raise NotImplementedError("write your pallas kernel here")


# speedup vs baseline: 1.2783x; 1.2783x over previous
import jax
import jax.numpy as jnp
import numpy as np
from jax.experimental import pallas as pl
from jax.experimental.pallas import tpu as pltpu
from jax._src.random import threefry2x32 as _threefry

_NV = 1000
_ROWS = 256
_NROWS = 65536
_TINY = np.float32(np.finfo(np.float32).tiny)


def _body(z_ref, out_ref):
    rows = z_ref.shape[0]
    base = pl.program_id(0) * (rows * _NV)
    row = jax.lax.broadcasted_iota(jnp.int32, (rows, _NV), 0)
    col = jax.lax.broadcasted_iota(jnp.int32, (rows, _NV), 1)
    cnt = (base + row * _NV + col).astype(jnp.uint32)
    k1 = jnp.full((1, 1), 0, jnp.uint32)
    k2 = jnp.full((1, 1), 42, jnp.uint32)
    o1, o2 = _threefry.threefry2x32_p.bind(k1, k2, jnp.zeros_like(cnt), cnt)
    bits = o1 ^ o2
    fb = (bits >> jnp.uint32(9)) | jnp.uint32(0x3F800000)
    f = jax.lax.bitcast_convert_type(fb, jnp.float32) - jnp.float32(1.0)
    u = jnp.maximum(f, _TINY)
    g = -jnp.log(-jnp.log(u))
    score = z_ref[...] + g
    mx = jnp.max(score, axis=1, keepdims=True)
    idx = jnp.min(jnp.where(score == mx, col, _NV), axis=1, keepdims=True)
    out_ref[...] = idx


def kernel(z):
    b = z.shape[0]
    zr = z.reshape(_NROWS, _NV)
    out = pl.pallas_call(
        _body,
        grid=(_NROWS // _ROWS,),
        in_specs=[pl.BlockSpec((_ROWS, _NV), lambda i: (i, 0))],
        out_specs=pl.BlockSpec((_ROWS, 1), lambda i: (i, 0)),
        out_shape=jax.ShapeDtypeStruct((_NROWS, 1), jnp.int32),
        compiler_params=pltpu.CompilerParams(
            dimension_semantics=("parallel",),
        ),
    )(zr)
    return out.reshape(b, _NROWS // b)
